# Initial kernel scaffold; baseline (speedup 1.0000x reference)
import jax
import jax.numpy as jnp
from jax.experimental import pallas as pl


def kernel(xh, node_mask, edge_mask, eps_x, eps_h, params):
    raise NotImplementedError("write your pallas kernel here")


# speedup vs baseline: 13.3691x; 13.3691x over previous
import jax
import jax.numpy as jnp
from jax.experimental import pallas as pl
from jax.experimental.pallas import tpu as pltpu


def _silu(v):
    return v * jax.nn.sigmoid(v)


def _mm(a, w16):
    return jax.lax.dot_general(
        a.astype(jnp.bfloat16), w16, (((a.ndim - 1,), (0,)), ((), ())),
        preferred_element_type=jnp.float32)


def kernel(xh, node_mask, edge_mask, eps_x, eps_h, params):
    BS, N, F = xh.shape
    D = 3
    layers = params["layers"]
    NL = len(layers)
    HID = params["embed_W"].shape[1]
    LAT = params["vae"]["xdW1"].shape[0]

    xs = xh[..., :D]
    masked = xs * node_mask
    n = jnp.sum(node_mask, axis=1, keepdims=True)
    mean = jnp.sum(masked, axis=1, keepdims=True) / n
    x_in3 = xs - mean * node_mask

    ex = eps_x.reshape(BS, N, LAT)
    eh = eps_h.reshape(BS, N, LAT)

    def st(k):
        return jnp.stack([lp[k] for lp in layers])

    def stb(k):
        return jnp.stack([lp[k].reshape(1, -1) for lp in layers])

    vp = params["vae"]

    def b2(b):
        return b.reshape(1, -1)

    def c16(a):
        return a.astype(jnp.bfloat16)

    blocked = [
        (xh, (1, N, F)),
        (x_in3, (1, N, D)),
        (node_mask, (1, N, 1)),
        (edge_mask, (1, N * N, 1)),
        (ex, (1, N, LAT)),
        (eh, (1, N, LAT)),
    ]
    full = [
        c16(params["embed_W"]), b2(params["embed_b"]),
        c16(st("eW1")), stb("eb1"), c16(st("eW2")), stb("eb2"),
        c16(st("cW1")), stb("cb1"), c16(st("cW2")),
        jnp.stack([lp["cb2"].reshape(1, 1) for lp in layers]),
        c16(st("nW1")), stb("nb1"), c16(st("nW2")), stb("nb2"),
        c16(vp["xeW1"]), b2(vp["xeb1"]), c16(vp["xeW2"]), b2(vp["xeb2"]),
        c16(vp["heW1"]), b2(vp["heb1"]), c16(vp["heW2"]), b2(vp["heb2"]),
        c16(vp["xdW1"]), b2(vp["xdb1"]), c16(vp["xdW2"]), b2(vp["xdb2"]),
        c16(vp["hdW1"]), b2(vp["hdb1"]), c16(vp["hdW2"]), b2(vp["hdb2"]),
    ]

    def body(xh_r, xin_r, nm_r, em_r, ex_r, eh_r,
             embW_r, embb_r,
             eW1_r, eb1_r, eW2_r, eb2_r,
             cW1_r, cb1_r, cW2_r, cb2_r,
             nW1_r, nb1_r, nW2_r, nb2_r,
             xeW1_r, xeb1_r, xeW2_r, xeb2_r,
             heW1_r, heb1_r, heW2_r, heb2_r,
             xdW1_r, xdb1_r, xdW2_r, xdb2_r,
             hdW1_r, hdb1_r, hdW2_r, hdb2_r,
             out_r, part_r):
        h0 = xh_r[0][:, D:]
        x_in = xin_r[0]
        nm = nm_r[0]
        em = em_r[0]
        nsum = jnp.sum(nm)

        h = _mm(h0, embW_r[...]) + embb_r[...]
        x = x_in
        agg_x = jnp.zeros((N, D), jnp.float32)

        for l in range(NL):
            hi = jnp.broadcast_to(h[:, None, :], (N, N, HID)).reshape(N * N, HID)
            hj = jnp.broadcast_to(h[None, :, :], (N, N, HID)).reshape(N * N, HID)
            xr = jnp.broadcast_to(x[:, None, :], (N, N, D)).reshape(N * N, D)
            xc = jnp.broadcast_to(x[None, :, :], (N, N, D)).reshape(N * N, D)
            diff = xr - xc
            d2 = jnp.sum(diff ** 2, axis=1, keepdims=True)
            cc = jnp.concatenate([hi, hj, d2], axis=1)
            m1 = _silu(_mm(cc, eW1_r[l]) + eb1_r[l])
            m2 = _silu(_mm(m1, eW2_r[l]) + eb2_r[l])
            m3 = m2 * em
            wp = _silu(_mm(m3, cW1_r[l]) + cb1_r[l])
            wgt = jnp.tanh(_mm(wp, cW2_r[l]) + cb2_r[l])
            pf = (diff * wgt).reshape(N, N, D)
            m3r = m3.reshape(N, N, HID)
            agg_x = jnp.zeros((N, D), jnp.float32)
            agg_m = jnp.zeros((N, HID), jnp.float32)
            for j in range(N):
                agg_x = agg_x + pf[:, j, :]
                agg_m = agg_m + m3r[:, j, :]
            x = (x + agg_x / float(N)) * nm
            cc2 = jnp.concatenate([h, agg_m], axis=1)
            t = _silu(_mm(cc2, nW1_r[l]) + nb1_r[l])
            h = (h + (_mm(t, nW2_r[l]) + nb2_r[l])) * nm

        v1 = x - x_in
        v2 = agg_x

        def nrm(v):
            return v / (jnp.sqrt(jnp.sum(v * v, axis=1, keepdims=True)) + 1e-6)

        e1 = nrm(v1)
        u2 = v2 - jnp.sum(v2 * e1, axis=1, keepdims=True) * e1
        e2 = nrm(u2)
        a1, b1, c1 = e1[:, 0:1], e1[:, 1:2], e1[:, 2:3]
        a2, b2_, c2 = e2[:, 0:1], e2[:, 1:2], e2[:, 2:3]
        e3 = jnp.concatenate(
            [b1 * c2 - c1 * b2_, c1 * a2 - a1 * c2, a1 * b2_ - b1 * a2],
            axis=1)
        x_inv = jnp.concatenate(
            [jnp.sum(x * e1, axis=1, keepdims=True),
             jnp.sum(x * e2, axis=1, keepdims=True),
             jnp.sum(x * e3, axis=1, keepdims=True)], axis=1)

        xs_ = _mm(_silu(_mm(x_inv, xeW1_r[...]) + xeb1_r[...]),
                  xeW2_r[...]) + xeb2_r[...]
        hs_ = _mm(_silu(_mm(h0, heW1_r[...]) + heb1_r[...]),
                  heW2_r[...]) + heb2_r[...]
        x_mu, x_lv = xs_[:, :LAT], xs_[:, LAT:]
        h_mu, h_lv = hs_[:, :LAT], hs_[:, LAT:]
        zx = x_mu + jnp.exp(0.5 * x_lv) * ex_r[0]
        zh = h_mu + jnp.exp(0.5 * h_lv) * eh_r[0]
        x_rec = _mm(_silu(_mm(zx, xdW1_r[...]) + xdb1_r[...]),
                    xdW2_r[...]) + xdb2_r[...]
        h_rec = _mm(_silu(_mm(zh, hdW1_r[...]) + hdb1_r[...]),
                    hdW2_r[...]) + hdb2_r[...]
        xkl = 0.5 * jnp.sum(x_mu ** 2 + jnp.exp(x_lv) - 1.0 - x_lv)
        hkl = 0.5 * jnp.sum(h_mu ** 2 + jnp.exp(h_lv) - 1.0 - h_lv)

        xfe = x_rec[:, 0:1] * e1 + x_rec[:, 1:2] * e2 + x_rec[:, 2:3] * e3
        vel_pre = xfe * nm
        nanflag = jnp.any(jnp.isnan(vel_pre)).astype(jnp.float32)
        mv = jnp.sum(vel_pre * nm, axis=0, keepdims=True) / nsum
        vel_c = vel_pre - mv * nm
        sxr = jnp.sum(nm * (vel_c - x_in) ** 2)
        sx2 = jnp.sum(nm * x_in ** 2)
        shr = jnp.sum(nm * (h_rec - h0) ** 2)

        out_r[0] = jnp.concatenate([vel_c, h_rec], axis=1)
        part_r[0] = jnp.concatenate(
            [v.reshape(1, 1) for v in
             [nanflag, sxr, sx2, shr, xkl, hkl, nsum, jnp.zeros(())]],
            axis=1)

    in_specs = (
        [pl.BlockSpec(bs, lambda g: (g, 0, 0)) for _, bs in blocked]
        + [pl.BlockSpec(a.shape, lambda g, nd=a.ndim: (0,) * nd)
           for a in full])
    out_specs = [
        pl.BlockSpec((1, N, F), lambda g: (g, 0, 0)),
        pl.BlockSpec((1, 1, 8), lambda g: (g, 0, 0)),
    ]
    out_shape = [
        jax.ShapeDtypeStruct((BS, N, F), jnp.float32),
        jax.ShapeDtypeStruct((BS, 1, 8), jnp.float32),
    ]

    out, parts = pl.pallas_call(
        body,
        grid=(BS,),
        in_specs=in_specs,
        out_specs=out_specs,
        out_shape=out_shape,
        compiler_params=pltpu.CompilerParams(
            dimension_semantics=("parallel",)),
    )(*([a for a, _ in blocked] + full))

    p = parts.reshape(BS, 8)
    nan_any = jnp.max(p[:, 0]) > 0.0
    snm = jnp.sum(p[:, 6])
    x_recon = jnp.where(nan_any, jnp.sum(p[:, 2]), jnp.sum(p[:, 1])) / snm
    h_recon = jnp.sum(p[:, 3]) / snm
    kl = (jnp.sum(p[:, 4]) + jnp.sum(p[:, 5])) / float(BS * N)
    recon = x_recon + h_recon
    total = recon + 1e-05 * kl
    vel = jnp.where(nan_any, jnp.zeros_like(out[..., :D]), out[..., :D])
    model_output = jnp.concatenate([vel, out[..., D:]], axis=-1)
    return model_output, total, recon, kl
